# baseline (speedup 1.0000x reference)
import jax
import jax.numpy as jnp
from jax import lax
from jax.experimental import pallas as pl
from jax.experimental.pallas import tpu as pltpu
from jax.experimental.pallas import tpu_sc as plsc

D_MODEL = 768
HALF = D_MODEL // 2
N_ROWS = 4 * 2048
TAB = 100 * HALF

_info = plsc.get_sparse_core_info()
NC = _info.num_cores
NS = _info.num_subcores
NW = NC * NS
ROWS_PER_W = N_ROWS // NW
SUB = 8
SUB_ROWS = ROWS_PER_W // SUB
LANES = 16


def _sc_body(x2, bflat, pe_hf, pe_wf, out,
             b_v, cy_v, cx_v,
             x_v0, x_v1, th_v, tw_v,
             sem_i0, sem_i1, sem_o0, sem_o1, sem_t):
    wid = lax.axis_index("s") * NC + lax.axis_index("c")
    base = wid * ROWS_PER_W

    t_d = (pltpu.async_copy(pe_hf, th_v, sem_t),
           pltpu.async_copy(pe_wf, tw_v, sem_t))

    pltpu.sync_copy(bflat.at[pl.ds(base * 4, ROWS_PER_W * 4)], b_v)

    for i in range(ROWS_PER_W // LANES):
        sl = pl.ds(i * LANES, LANES)
        off = lax.iota(jnp.int32, LANES) * 4 + (i * LANES * 4)
        v0 = plsc.load_gather(b_v, [off])
        v1 = plsc.load_gather(b_v, [off + 1])
        v2 = plsc.load_gather(b_v, [off + 2])
        v3 = plsc.load_gather(b_v, [off + 3])
        cx = ((v0 + v2) * 0.5 * 99.0).astype(jnp.int32)
        cy = ((v1 + v3) * 0.5 * 99.0).astype(jnp.int32)
        cx_v[sl] = jnp.minimum(jnp.maximum(cx, 0), 99) * HALF
        cy_v[sl] = jnp.minimum(jnp.maximum(cy, 0), 99) * HALF

    xb = (x_v0, x_v1)
    si = (sem_i0, sem_i1)
    so = (sem_o0, sem_o1)
    in_d = [None, None]
    out_d = [None, None]

    def fire(s):
        b = s & 1
        rsl = pl.ds(base + s * SUB_ROWS, SUB_ROWS)
        in_d[b] = pltpu.async_copy(x2.at[rsl, :], xb[b], si[b])

    fire(0)
    for d in t_d:
        d.wait()
    for s in range(SUB):
        b = s & 1
        if s + 1 < SUB:
            if out_d[1 - b] is not None:
                out_d[1 - b].wait()
            fire(s + 1)
        in_d[b].wait()

        x_v = xb[b]
        for g in range(SUB_ROWS // LANES):
            gsl = pl.ds(s * SUB_ROWS + g * LANES, LANES)
            gy = cy_v[gsl]
            gx = cx_v[gsl]

            @plsc.parallel_loop(0, LANES, unroll=2)
            def _row(l):
                lv = jnp.full((LANES,), l, dtype=jnp.int32)
                yb = gy.at[lv].get(mode="promise_in_bounds")
                wb = gx.at[lv].get(mode="promise_in_bounds")
                r = g * LANES + l
                for c in range(HALF // LANES):
                    col = lax.iota(jnp.int32, LANES) + (c * LANES)
                    ph = plsc.load_gather(th_v, [yb + col])
                    pw = plsc.load_gather(tw_v, [wb + col])
                    plsc.addupdate(x_v.at[r, pl.ds(c * LANES, LANES)], ph)
                    plsc.addupdate(
                        x_v.at[r, pl.ds(HALF + c * LANES, LANES)], pw)

        rsl = pl.ds(base + s * SUB_ROWS, SUB_ROWS)
        out_d[b] = pltpu.async_copy(x_v, out.at[rsl, :], so[b])

    out_d[0].wait()
    out_d[1].wait()


def kernel(x, boxes, pe_h, pe_w):
    x2 = x.reshape(N_ROWS, D_MODEL)
    bflat = boxes.reshape(N_ROWS * 4)

    mesh = plsc.VectorSubcoreMesh(core_axis_name="c", subcore_axis_name="s")
    run = pl.kernel(
        _sc_body,
        mesh=mesh,
        compiler_params=pltpu.CompilerParams(needs_layout_passes=False),
        out_type=jax.ShapeDtypeStruct((N_ROWS, D_MODEL), jnp.float32),
        scratch_types=[
            pltpu.VMEM((ROWS_PER_W * 4,), jnp.float32),
            pltpu.VMEM((ROWS_PER_W,), jnp.int32),
            pltpu.VMEM((ROWS_PER_W,), jnp.int32),
            pltpu.VMEM((SUB_ROWS, D_MODEL), jnp.float32),
            pltpu.VMEM((SUB_ROWS, D_MODEL), jnp.float32),
            pltpu.VMEM((TAB,), jnp.float32),
            pltpu.VMEM((TAB,), jnp.float32),
            pltpu.SemaphoreType.DMA,
            pltpu.SemaphoreType.DMA,
            pltpu.SemaphoreType.DMA,
            pltpu.SemaphoreType.DMA,
            pltpu.SemaphoreType.DMA,
        ],
    )
    out2 = run(x2, bflat, pe_h.reshape(-1), pe_w.reshape(-1))
    return out2.reshape(x.shape)


# speedup vs baseline: 1.0550x; 1.0550x over previous
import jax
import jax.numpy as jnp
from jax import lax
from jax.experimental import pallas as pl
from jax.experimental.pallas import tpu as pltpu
from jax.experimental.pallas import tpu_sc as plsc

D_MODEL = 768
HALF = D_MODEL // 2
N_ROWS = 4 * 2048
TAB = 100 * HALF

_info = plsc.get_sparse_core_info()
NC = _info.num_cores
NS = _info.num_subcores
NW = NC * NS
ROWS_PER_W = N_ROWS // NW
SUB = 8
SUB_ROWS = ROWS_PER_W // SUB
LANES = 16


def _sc_body(x2, b0, b1, b2, b3, pe_hf, pe_wf, out,
             b0_v, b1_v, b2_v, b3_v, cy_v, cx_v,
             x_v0, x_v1, th_v, tw_v,
             sem_i0, sem_i1, sem_o0, sem_o1, sem_t):
    wid = lax.axis_index("s") * NC + lax.axis_index("c")
    base = wid * ROWS_PER_W

    t_d = (pltpu.async_copy(pe_hf, th_v, sem_t),
           pltpu.async_copy(pe_wf, tw_v, sem_t))

    pltpu.sync_copy(b0.at[pl.ds(base, ROWS_PER_W)], b0_v)
    pltpu.sync_copy(b1.at[pl.ds(base, ROWS_PER_W)], b1_v)
    pltpu.sync_copy(b2.at[pl.ds(base, ROWS_PER_W)], b2_v)
    pltpu.sync_copy(b3.at[pl.ds(base, ROWS_PER_W)], b3_v)

    for i in range(ROWS_PER_W // LANES):
        sl = pl.ds(i * LANES, LANES)
        cx = ((b0_v[sl] + b2_v[sl]) * 0.5 * 99.0).astype(jnp.int32)
        cy = ((b1_v[sl] + b3_v[sl]) * 0.5 * 99.0).astype(jnp.int32)
        cx_v[sl] = jnp.minimum(jnp.maximum(cx, 0), 99) * HALF
        cy_v[sl] = jnp.minimum(jnp.maximum(cy, 0), 99) * HALF

    xb = (x_v0, x_v1)
    si = (sem_i0, sem_i1)
    so = (sem_o0, sem_o1)
    in_d = [None, None]
    out_d = [None, None]

    def fire(s):
        b = s & 1
        rsl = pl.ds(base + s * SUB_ROWS, SUB_ROWS)
        in_d[b] = pltpu.async_copy(x2.at[rsl, :], xb[b], si[b])

    fire(0)
    for d in t_d:
        d.wait()
    for s in range(SUB):
        b = s & 1
        if s + 1 < SUB:
            if out_d[1 - b] is not None:
                out_d[1 - b].wait()
            fire(s + 1)
        in_d[b].wait()

        x_v = xb[b]
        for g in range(SUB_ROWS // LANES):
            gsl = pl.ds(s * SUB_ROWS + g * LANES, LANES)
            gy = cy_v[gsl]
            gx = cx_v[gsl]

            @plsc.parallel_loop(0, LANES, unroll=2)
            def _row(l):
                lv = jnp.full((LANES,), l, dtype=jnp.int32)
                yb = gy.at[lv].get(mode="promise_in_bounds")
                wb = gx.at[lv].get(mode="promise_in_bounds")
                r = g * LANES + l
                for c in range(HALF // LANES):
                    col = lax.iota(jnp.int32, LANES) + (c * LANES)
                    ph = plsc.load_gather(th_v, [yb + col])
                    pw = plsc.load_gather(tw_v, [wb + col])
                    plsc.addupdate(x_v.at[r, pl.ds(c * LANES, LANES)], ph)
                    plsc.addupdate(
                        x_v.at[r, pl.ds(HALF + c * LANES, LANES)], pw)

        rsl = pl.ds(base + s * SUB_ROWS, SUB_ROWS)
        out_d[b] = pltpu.async_copy(x_v, out.at[rsl, :], so[b])

    out_d[0].wait()
    out_d[1].wait()


def kernel(x, boxes, pe_h, pe_w):
    x2 = x.reshape(N_ROWS, D_MODEL)
    bf = boxes.reshape(N_ROWS, 4)
    b0 = bf[:, 0]
    b1 = bf[:, 1]
    b2 = bf[:, 2]
    b3 = bf[:, 3]

    mesh = plsc.VectorSubcoreMesh(core_axis_name="c", subcore_axis_name="s")
    run = pl.kernel(
        _sc_body,
        mesh=mesh,
        compiler_params=pltpu.CompilerParams(needs_layout_passes=False),
        out_type=jax.ShapeDtypeStruct((N_ROWS, D_MODEL), jnp.float32),
        scratch_types=[
            pltpu.VMEM((ROWS_PER_W,), jnp.float32),
            pltpu.VMEM((ROWS_PER_W,), jnp.float32),
            pltpu.VMEM((ROWS_PER_W,), jnp.float32),
            pltpu.VMEM((ROWS_PER_W,), jnp.float32),
            pltpu.VMEM((ROWS_PER_W,), jnp.int32),
            pltpu.VMEM((ROWS_PER_W,), jnp.int32),
            pltpu.VMEM((SUB_ROWS, D_MODEL), jnp.float32),
            pltpu.VMEM((SUB_ROWS, D_MODEL), jnp.float32),
            pltpu.VMEM((TAB,), jnp.float32),
            pltpu.VMEM((TAB,), jnp.float32),
            pltpu.SemaphoreType.DMA,
            pltpu.SemaphoreType.DMA,
            pltpu.SemaphoreType.DMA,
            pltpu.SemaphoreType.DMA,
            pltpu.SemaphoreType.DMA,
        ],
    )
    out2 = run(x2, b0, b1, b2, b3, pe_h.reshape(-1), pe_w.reshape(-1))
    return out2.reshape(x.shape)
